# baseline (speedup 1.0000x reference)
import functools

import jax
import jax.numpy as jnp
from jax import lax
from jax.experimental import pallas as pl
from jax.experimental.pallas import tpu as pltpu
from jax.experimental.pallas import tpu_sc as plsc

F = 32
P = 512
NP = 3
M = 262144
NW = 32
B = 128
G = 16

PLANE_XY = ((0, 1), (0, 2), (2, 1))


def _tap_setup(x, y, plane_base):
    ix = (x + 1.0) * (P // 2) - 0.5
    iy = (y + 1.0) * (P // 2) - 0.5
    fxi = (ix + P).astype(jnp.int32) - P
    fyi = (iy + P).astype(jnp.int32) - P
    wx1 = ix - fxi.astype(jnp.float32)
    wy1 = iy - fyi.astype(jnp.float32)
    wx0 = 1.0 - wx1
    wy0 = 1.0 - wy1
    ax0 = jnp.where((fxi >= 0) & (fxi <= P - 1), wx0, 0.0)
    ax1 = jnp.where((fxi + 1 >= 0) & (fxi + 1 <= P - 1), wx1, 0.0)
    ay0 = jnp.where((fyi >= 0) & (fyi <= P - 1), wy0, 0.0)
    ay1 = jnp.where((fyi + 1 >= 0) & (fyi + 1 <= P - 1), wy1, 0.0)
    cx0 = jnp.clip(fxi, 0, P - 1)
    cx1 = jnp.clip(fxi + 1, 0, P - 1)
    cy0 = jnp.clip(fyi, 0, P - 1)
    cy1 = jnp.clip(fyi + 1, 0, P - 1)
    r0 = plane_base + cy0 * P
    r1 = plane_base + cy1 * P
    idxs = (r0 + cx0, r0 + cx1, r1 + cx0, r1 + cx1)
    ws = (ax0 * ay0, ax1 * ay0, ax0 * ay1, ax1 * ay1)
    return idxs, ws


def _sc_body(coords_hbm, table_hbm, out_hbm,
             cv, idx_v, w_v, rows_v, out_v, sems):
    wid = lax.axis_index("s") * 2 + lax.axis_index("c")
    per_w = M // NW
    mbase = wid * per_w
    n_chunks = per_w // B

    def stage_and_fire(g, buf):
        base = mbase + g * B
        pltpu.sync_copy(coords_hbm.at[pl.ds(base, B)], cv.at[buf])
        for i in range(B // G):
            rows16 = lax.iota(jnp.int32, G) + i * G
            comp = [plsc.load_gather(cv.at[buf],
                                     [rows16, jnp.full((G,), c, jnp.int32)])
                    for c in range(3)]
            for p, (cxs, cys) in enumerate(PLANE_XY):
                sl = pl.ds(i * G, G)
                idxs, ws = _tap_setup(comp[cxs], comp[cys], p * P * P)
                for t in range(4):
                    idx_v[buf, 4 * p + t, sl] = idxs[t]
                    w_v[buf, 4 * p + t, sl] = ws[t]
        for t in range(4 * NP):
            pltpu.async_copy(table_hbm.at[idx_v.at[buf, t]],
                             rows_v.at[buf, t], sems.at[buf])

    def drain(buf):
        for t in range(4 * NP):
            pltpu.make_async_copy(table_hbm.at[idx_v.at[buf, t]],
                                  rows_v.at[buf, t], sems.at[buf]).wait()

    def compute(g, buf):
        def group(gi, carry2):
            gb = gi * G
            wvec = [w_v[buf, t, pl.ds(gb, G)] for t in range(4 * NP)]
            for j in range(G):
                b = gb + j
                for p in range(NP):
                    acc_lo = None
                    acc_hi = None
                    for t in range(4):
                        w = wvec[4 * p + t][j]
                        row = rows_v[buf, 4 * p + t, b, :]
                        lo, hi = plsc.unpack(row,
                                             format=plsc.PackFormat.INTERLEAVED)
                        if acc_lo is None:
                            acc_lo = w * lo
                            acc_hi = w * hi
                        else:
                            acc_lo = acc_lo + w * lo
                            acc_hi = acc_hi + w * hi
                    out_v[b, pl.ds(p * F, G)] = acc_lo
                    out_v[b, pl.ds(p * F + G, G)] = acc_hi
            return carry2

        lax.fori_loop(0, B // G, group, 0)
        pltpu.sync_copy(out_v, out_hbm.at[0, pl.ds(mbase + g * B, B)])

    stage_and_fire(0, 0)

    def pair(gg, carry):
        for par in range(2):
            g = 2 * gg + par
            drain(par)
            nxt = g + 1

            @pl.when(nxt < n_chunks)
            def _():
                stage_and_fire(nxt, 1 - par)

            compute(g, par)
        return carry

    lax.fori_loop(0, n_chunks // 2, pair, 0)


_sc_call = functools.partial(
    pl.kernel,
    mesh=plsc.VectorSubcoreMesh(core_axis_name="c", subcore_axis_name="s"),
    out_type=jax.ShapeDtypeStruct((1, M, NP * F), jnp.float32),
    scratch_types=[
        pltpu.VMEM((2, B, 3), jnp.float32),
        pltpu.VMEM((2, 4 * NP, B), jnp.int32),
        pltpu.VMEM((2, 4 * NP, B), jnp.float32),
        pltpu.VMEM((2, 4 * NP, B, F), jnp.bfloat16),
        pltpu.VMEM((B, NP * F), jnp.float32),
        pltpu.SemaphoreType.DMA((2,)),
    ],
    compiler_params=pltpu.CompilerParams(use_tc_tiling_on_sc=False,
                                         needs_layout_passes=False),
)(_sc_body)


def kernel(coords, tplanes):
    c = lax.reduce_precision(coords[0], exponent_bits=8, mantissa_bits=7)
    table = (tplanes[0]
             .reshape(NP, 2, F // 2, P, P)
             .transpose(0, 3, 4, 2, 1)
             .reshape(NP * P * P, F)
             .astype(jnp.bfloat16))
    return _sc_call(c, table)


# speedup vs baseline: 1.2023x; 1.2023x over previous
import functools

import numpy as np

import jax
import jax.numpy as jnp
from jax import lax
from jax.experimental import pallas as pl
from jax.experimental.pallas import tpu as pltpu
from jax.experimental.pallas import tpu_sc as plsc

F = 32
P = 512
NP = 3
M = 262144
NW = 32
B = 128
G = 16
V = NP * P * P

PLANE_XY = ((0, 1), (0, 2), (2, 1))

_IPERM = np.zeros((F, F), np.float32)
for _c in range(F):
    _IPERM[_c, 2 * (_c % (F // 2)) + _c // (F // 2)] = 1.0
_IPERM = jnp.asarray(_IPERM)


def _pack_body(tp_ref, iperm_ref, out_ref):
    x = tp_ref[0].reshape(F, 8 * P)
    xt = lax.dot_general(x, iperm_ref[...],
                         dimension_numbers=(((0,), (0,)), ((), ())),
                         preferred_element_type=jnp.float32)
    out_ref[...] = xt.astype(jnp.bfloat16)


_pack_table = pl.pallas_call(
    _pack_body,
    grid=(NP, P // 8),
    in_specs=[pl.BlockSpec((1, F, 8, P), lambda i, j: (i, 0, j, 0)),
              pl.BlockSpec((F, F), lambda i, j: (0, 0))],
    out_specs=pl.BlockSpec((8 * P, F), lambda i, j: (i * (P // 8) + j, 0)),
    out_shape=jax.ShapeDtypeStruct((V, F), jnp.bfloat16),
)


def _tap_setup(x, y, plane_base):
    ix = (x + 1.0) * (P // 2) - 0.5
    iy = (y + 1.0) * (P // 2) - 0.5
    fxi = (ix + P).astype(jnp.int32) - P
    fyi = (iy + P).astype(jnp.int32) - P
    wx1 = ix - fxi.astype(jnp.float32)
    wy1 = iy - fyi.astype(jnp.float32)
    wx0 = 1.0 - wx1
    wy0 = 1.0 - wy1
    ax0 = jnp.where((fxi >= 0) & (fxi <= P - 1), wx0, 0.0)
    ax1 = jnp.where((fxi + 1 >= 0) & (fxi + 1 <= P - 1), wx1, 0.0)
    ay0 = jnp.where((fyi >= 0) & (fyi <= P - 1), wy0, 0.0)
    ay1 = jnp.where((fyi + 1 >= 0) & (fyi + 1 <= P - 1), wy1, 0.0)
    cx0 = jnp.clip(fxi, 0, P - 1)
    cx1 = jnp.clip(fxi + 1, 0, P - 1)
    cy0 = jnp.clip(fyi, 0, P - 1)
    cy1 = jnp.clip(fyi + 1, 0, P - 1)
    r0 = plane_base + cy0 * P
    r1 = plane_base + cy1 * P
    idxs = (r0 + cx0, r0 + cx1, r1 + cx0, r1 + cx1)
    ws = (ax0 * ay0, ax1 * ay0, ax0 * ay1, ax1 * ay1)
    return idxs, ws


def _sc_body(coords_hbm, table_hbm, out_hbm,
             cv, idx_v, w_v, rows_v, out_v, sems):
    wid = lax.axis_index("s") * 2 + lax.axis_index("c")
    per_w = M // NW
    mbase = wid * per_w
    n_chunks = per_w // B

    def stage_and_fire(g, buf):
        base = mbase + g * B
        pltpu.sync_copy(coords_hbm.at[pl.ds(base * 3, B * 3)], cv.at[buf])
        for i in range(B // G):
            pos3 = (lax.iota(jnp.int32, G) + i * G) * 3
            comp = [plsc.load_gather(cv.at[buf], [pos3 + c]) for c in range(3)]
            for p, (cxs, cys) in enumerate(PLANE_XY):
                sl = pl.ds(i * G, G)
                idxs, ws = _tap_setup(comp[cxs], comp[cys], p * P * P)
                for t in range(4):
                    idx_v[buf, 4 * p + t, sl] = idxs[t]
                    w_v[buf, 4 * p + t, sl] = ws[t]
        for t in range(4 * NP):
            pltpu.async_copy(table_hbm.at[idx_v.at[buf, t]],
                             rows_v.at[buf, t], sems.at[buf])

    def drain(buf):
        for t in range(4 * NP):
            pltpu.make_async_copy(table_hbm.at[idx_v.at[buf, t]],
                                  rows_v.at[buf, t], sems.at[buf]).wait()

    def compute(g, buf):
        def group(gi, carry2):
            gb = gi * G
            wvec = [w_v[buf, t, pl.ds(gb, G)] for t in range(4 * NP)]
            for j in range(G):
                b = gb + j
                o = b * (NP * F)
                for p in range(NP):
                    acc_lo = None
                    acc_hi = None
                    for t in range(4):
                        w = wvec[4 * p + t][j]
                        row = rows_v[buf, 4 * p + t, b, :]
                        lo, hi = plsc.unpack(row,
                                             format=plsc.PackFormat.INTERLEAVED)
                        if acc_lo is None:
                            acc_lo = w * lo
                            acc_hi = w * hi
                        else:
                            acc_lo = acc_lo + w * lo
                            acc_hi = acc_hi + w * hi
                    out_v[pl.ds(o + p * F, G)] = acc_lo
                    out_v[pl.ds(o + p * F + G, G)] = acc_hi
            return carry2

        lax.fori_loop(0, B // G, group, 0)
        pltpu.sync_copy(out_v,
                        out_hbm.at[pl.ds((mbase + g * B) * (NP * F),
                                         B * NP * F)])

    stage_and_fire(0, 0)

    def pair(gg, carry):
        for par in range(2):
            g = 2 * gg + par
            drain(par)
            nxt = g + 1

            @pl.when(nxt < n_chunks)
            def _():
                stage_and_fire(nxt, 1 - par)

            compute(g, par)
        return carry

    lax.fori_loop(0, n_chunks // 2, pair, 0)


_sc_call = functools.partial(
    pl.kernel,
    mesh=plsc.VectorSubcoreMesh(core_axis_name="c", subcore_axis_name="s"),
    out_type=jax.ShapeDtypeStruct((M * NP * F,), jnp.float32),
    scratch_types=[
        pltpu.VMEM((2, B * 3), jnp.float32),
        pltpu.VMEM((2, 4 * NP, B), jnp.int32),
        pltpu.VMEM((2, 4 * NP, B), jnp.float32),
        pltpu.VMEM((2, 4 * NP, B, F), jnp.bfloat16),
        pltpu.VMEM((B * NP * F,), jnp.float32),
        pltpu.SemaphoreType.DMA((2,)),
    ],
    compiler_params=pltpu.CompilerParams(use_tc_tiling_on_sc=False,
                                         needs_layout_passes=False),
)(_sc_body)


def kernel(coords, tplanes):
    c = lax.reduce_precision(coords.reshape(M * 3), exponent_bits=8,
                             mantissa_bits=7)
    table = _pack_table(tplanes[0], _IPERM)
    out = _sc_call(c, table)
    return out.reshape(1, M, NP * F)


# speedup vs baseline: 1.3265x; 1.1033x over previous
import functools

import jax
import jax.numpy as jnp
from jax import lax
from jax.experimental import pallas as pl
from jax.experimental.pallas import tpu as pltpu
from jax.experimental.pallas import tpu_sc as plsc

F = 32
P = 512
NP = 3
M = 262144
NW = 32
B = 128
G = 16

PLANE_XY = ((0, 1), (0, 2), (2, 1))


def _tap_setup(x, y, plane_base):
    ix = (x + 1.0) * (P // 2) - 0.5
    iy = (y + 1.0) * (P // 2) - 0.5
    fxi = (ix + P).astype(jnp.int32) - P
    fyi = (iy + P).astype(jnp.int32) - P
    wx1 = ix - fxi.astype(jnp.float32)
    wy1 = iy - fyi.astype(jnp.float32)
    wx0 = 1.0 - wx1
    wy0 = 1.0 - wy1
    ax0 = jnp.where((fxi >= 0) & (fxi <= P - 1), wx0, 0.0)
    ax1 = jnp.where((fxi + 1 >= 0) & (fxi + 1 <= P - 1), wx1, 0.0)
    ay0 = jnp.where((fyi >= 0) & (fyi <= P - 1), wy0, 0.0)
    ay1 = jnp.where((fyi + 1 >= 0) & (fyi + 1 <= P - 1), wy1, 0.0)
    cx0 = jnp.clip(fxi, 0, P - 1)
    cx1 = jnp.clip(fxi + 1, 0, P - 1)
    cy0 = jnp.clip(fyi, 0, P - 1)
    cy1 = jnp.clip(fyi + 1, 0, P - 1)
    r0 = plane_base + cy0 * P
    r1 = plane_base + cy1 * P
    idxs = (r0 + cx0, r0 + cx1, r1 + cx0, r1 + cx1)
    ws = (ax0 * ay0, ax1 * ay0, ax0 * ay1, ax1 * ay1)
    return idxs, ws


def _sc_body(xs_hbm, ys_hbm, zs_hbm, table_hbm, out_hbm,
             xv, yv, zv, idx_v, w_v, rows_v, out_v, sems):
    wid = lax.axis_index("s") * 2 + lax.axis_index("c")
    per_w = M // NW
    mbase = wid * per_w
    n_chunks = per_w // B

    def stage_and_fire(g, buf):
        base = mbase + g * B
        pltpu.sync_copy(xs_hbm.at[pl.ds(base, B)], xv.at[buf])
        pltpu.sync_copy(ys_hbm.at[pl.ds(base, B)], yv.at[buf])
        pltpu.sync_copy(zs_hbm.at[pl.ds(base, B)], zv.at[buf])
        comp = (xv.at[buf], yv.at[buf], zv.at[buf])
        for p, (cxs, cys) in enumerate(PLANE_XY):
            for i in range(B // G):
                sl = pl.ds(i * G, G)
                idxs, ws = _tap_setup(comp[cxs][sl], comp[cys][sl], p * P * P)
                for t in range(4):
                    idx_v[buf, 4 * p + t, sl] = idxs[t]
                    w_v[buf, 4 * p + t, sl] = ws[t]
        for t in range(4 * NP):
            pltpu.async_copy(table_hbm.at[idx_v.at[buf, t]],
                             rows_v.at[buf, t], sems.at[buf])

    def drain(buf):
        for t in range(4 * NP):
            pltpu.make_async_copy(table_hbm.at[idx_v.at[buf, t]],
                                  rows_v.at[buf, t], sems.at[buf]).wait()

    def compute(g, buf):
        def group(gi, carry2):
            gb = gi * G
            wvec = [w_v[buf, t, pl.ds(gb, G)] for t in range(4 * NP)]
            for j in range(G):
                b = gb + j
                for p in range(NP):
                    w0 = wvec[4 * p + 0][j]
                    w1 = wvec[4 * p + 1][j]
                    w2 = wvec[4 * p + 2][j]
                    w3 = wvec[4 * p + 3][j]
                    for h in range(F // G):
                        slh = pl.ds(h * G, G)
                        acc = w0 * rows_v[buf, 4 * p + 0, b, slh]
                        acc = acc + w1 * rows_v[buf, 4 * p + 1, b, slh]
                        acc = acc + w2 * rows_v[buf, 4 * p + 2, b, slh]
                        acc = acc + w3 * rows_v[buf, 4 * p + 3, b, slh]
                        out_v[b, pl.ds(p * F + h * G, G)] = acc
            return carry2

        lax.fori_loop(0, B // G, group, 0)
        pltpu.sync_copy(out_v, out_hbm.at[pl.ds(mbase + g * B, B)])

    stage_and_fire(0, 0)

    def pair(gg, carry):
        for par in range(2):
            g = 2 * gg + par
            drain(par)
            nxt = g + 1

            @pl.when(nxt < n_chunks)
            def _():
                stage_and_fire(nxt, 1 - par)

            compute(g, par)
        return carry

    lax.fori_loop(0, n_chunks // 2, pair, 0)


_sc_call = functools.partial(
    pl.kernel,
    mesh=plsc.VectorSubcoreMesh(core_axis_name="c", subcore_axis_name="s"),
    out_type=jax.ShapeDtypeStruct((M, NP * F), jnp.float32),
    scratch_types=[
        pltpu.VMEM((2, B), jnp.float32),
        pltpu.VMEM((2, B), jnp.float32),
        pltpu.VMEM((2, B), jnp.float32),
        pltpu.VMEM((2, 4 * NP, B), jnp.int32),
        pltpu.VMEM((2, 4 * NP, B), jnp.float32),
        pltpu.VMEM((2, 4 * NP, B, F), jnp.float32),
        pltpu.VMEM((B, NP * F), jnp.float32),
        pltpu.SemaphoreType.DMA((2,)),
    ],
    compiler_params=pltpu.CompilerParams(use_tc_tiling_on_sc=False),
)(_sc_body)


def kernel(coords, tplanes):
    c = lax.reduce_precision(coords[0], exponent_bits=8, mantissa_bits=7)
    xs = c[:, 0]
    ys = c[:, 1]
    zs = c[:, 2]
    table = jnp.transpose(tplanes[0], (0, 2, 3, 1)).reshape(NP * P * P, F)
    out = _sc_call(xs, ys, zs, table)
    return out.reshape(1, M, NP * F)
